# baseline (speedup 1.0000x reference)
import dataclasses
import functools

import jax
import jax.numpy as jnp
from jax import lax
from jax.experimental import pallas as pl
from jax.experimental.pallas import tpu as pltpu
from jax.experimental.pallas import tpu_sc as plsc

VOCAB_N = 1000000
DIM = 32
MVALS = 50
BATCH = 16384

NC = 2
NS = 16
NW = NC * NS
ROWS_PER_W = BATCH // NW
CHUNK_ROWS = 32
N_CHUNKS = ROWS_PER_W // CHUNK_ROWS
E = CHUNK_ROWS * MVALS
SB = 64
NSB = E // SB

_mesh = plsc.VectorSubcoreMesh(core_axis_name="c", subcore_axis_name="s")

_cp = pltpu.CompilerParams()
if "needs_layout_passes" in pltpu.CompilerParams.__dataclass_fields__:
    _cp = dataclasses.replace(_cp, needs_layout_passes=False)
if "use_tc_tiling_on_sc" in pltpu.CompilerParams.__dataclass_fields__:
    _cp = dataclasses.replace(_cp, use_tc_tiling_on_sc=False)


@functools.partial(
    pl.kernel,
    out_type=jax.ShapeDtypeStruct((BATCH, DIM), jnp.float32),
    mesh=_mesh,
    compiler_params=_cp,
    scratch_types=[
        pltpu.VMEM((NSB, 1, SB), jnp.int32),
        pltpu.VMEM((NSB, 1, SB), jnp.int32),
        pltpu.VMEM((E, DIM), jnp.float32),
        pltpu.VMEM((E, DIM), jnp.float32),
        pltpu.VMEM((CHUNK_ROWS, DIM), jnp.float32),
        pltpu.VMEM((1, DIM), jnp.float32),
        pltpu.SemaphoreType.DMA,
        pltpu.SemaphoreType.DMA,
    ],
)
def _sc_embed(idx_hbm, table_hbm, out_hbm,
              idxA, idxB, rowsA, rowsB, out_v, padrow, semA, semB):
    cid = lax.axis_index("c")
    sid = lax.axis_index("s")
    wid = cid * NS + sid

    iota = lax.iota(jnp.int32, 16)
    zero_i = jnp.zeros((16,), jnp.int32)
    vocab_v = jnp.full((16,), VOCAB_N, jnp.int32)
    mclamp = [jnp.minimum(iota + mb, MVALS - 1) for mb in (0, 16, 32, 48)]
    mvalid = [(iota + mb) < MVALS for mb in (0, 16, 32, 48)]
    csplat = [jnp.full((16,), c, jnp.int32) for c in range(DIM)]

    pltpu.sync_copy(table_hbm.at[pl.ds(VOCAB_N, 1)], padrow)
    pad0 = padrow[0, 0:16]
    pad1 = padrow[0, 16:32]

    def load_idx(ch, idx_v):
        sb_base = wid * (N_CHUNKS * NSB) + ch * NSB
        pltpu.sync_copy(idx_hbm.at[pl.ds(sb_base, NSB)], idx_v)

    def fire_gathers(idx_v, rows_v, sem):
        @pl.loop(0, NSB)
        def _fire(j):
            pltpu.async_copy(table_hbm.at[idx_v.at[j, 0]],
                             rows_v.at[pl.ds(j * SB, SB)], sem)

    def process(ch, idx_v, rows_v, sem):
        @pl.loop(0, NSB)
        def _drain(j):
            pltpu.make_async_copy(table_hbm.at[idx_v.at[j, 0]],
                                  rows_v.at[pl.ds(j * SB, SB)], sem).wait()

        @pl.loop(0, CHUNK_ROWS)
        def _row(r):
            r50 = jnp.full((16,), r * MVALS, jnp.int32)
            n_vec = zero_i
            nval_vec = zero_i
            for grp in range(4):
                ridx = r50 + mclamp[grp]
                rhi = ridx >> 6
                rlo = ridx & 63
                idxs = plsc.load_gather(idx_v, [rhi, zero_i, rlo])
                mx = plsc.load_gather(rows_v, [ridx, csplat[0]])
                for c in range(1, DIM):
                    mx = jnp.maximum(mx, plsc.load_gather(rows_v,
                                                          [ridx, csplat[c]]))
                valid = (idxs < vocab_v) & mvalid[grp]
                flag = valid & (mx > 0.0)
                n_vec = n_vec + plsc.all_reduce_population_count(flag)
                nval_vec = nval_vec + plsc.all_reduce_population_count(valid)
            e0 = r * MVALS
            s0 = rows_v[e0, 0:16]
            s1 = rows_v[e0, 16:32]
            for m in range(1, MVALS):
                s0 = s0 + rows_v[e0 + m, 0:16]
                s1 = s1 + rows_v[e0 + m, 16:32]
            npad = (MVALS - nval_vec).astype(jnp.float32)
            rec = 1.0 / jnp.maximum(n_vec.astype(jnp.float32), 1.0)
            out_v[r, 0:16] = (s0 - npad * pad0) * rec
            out_v[r, 16:32] = (s1 - npad * pad1) * rec

        rbase = wid * ROWS_PER_W + ch * CHUNK_ROWS
        pltpu.sync_copy(out_v, out_hbm.at[pl.ds(rbase, CHUNK_ROWS)])

    load_idx(0, idxA)
    fire_gathers(idxA, rowsA, semA)

    @pl.loop(0, N_CHUNKS // 2)
    def _pair(g):
        ch0 = 2 * g
        load_idx(ch0 + 1, idxB)
        fire_gathers(idxB, rowsB, semB)
        process(ch0, idxA, rowsA, semA)

        @pl.when(g < N_CHUNKS // 2 - 1)
        def _pf():
            load_idx(ch0 + 2, idxA)
            fire_gathers(idxA, rowsA, semA)

        process(ch0 + 1, idxB, rowsB, semB)


def kernel(indices, table):
    idx3 = indices.reshape(BATCH * MVALS // SB, 1, SB)
    out = _sc_embed(idx3, table)
    return out.reshape(BATCH, 1, DIM)


# speedup vs baseline: 1.7242x; 1.7242x over previous
import dataclasses
import functools

import jax
import jax.numpy as jnp
from jax import lax
from jax.experimental import pallas as pl
from jax.experimental.pallas import tpu as pltpu
from jax.experimental.pallas import tpu_sc as plsc

VOCAB_N = 1000000
DIM = 32
MVALS = 50
BATCH = 16384

NC = 2
NS = 16
NW = NC * NS
ROWS_PER_W = BATCH // NW
CHUNK_ROWS = 32
N_CHUNKS = ROWS_PER_W // CHUNK_ROWS
E = CHUNK_ROWS * MVALS
SB = 64
NSB = E // SB

_mesh = plsc.VectorSubcoreMesh(core_axis_name="c", subcore_axis_name="s")

_cp = pltpu.CompilerParams()
if "needs_layout_passes" in pltpu.CompilerParams.__dataclass_fields__:
    _cp = dataclasses.replace(_cp, needs_layout_passes=False)
if "use_tc_tiling_on_sc" in pltpu.CompilerParams.__dataclass_fields__:
    _cp = dataclasses.replace(_cp, use_tc_tiling_on_sc=False)


@functools.partial(
    pl.kernel,
    out_type=jax.ShapeDtypeStruct((BATCH, DIM), jnp.float32),
    mesh=_mesh,
    compiler_params=_cp,
    scratch_types=[
        pltpu.VMEM((NSB, 1, SB), jnp.int32),
        pltpu.VMEM((NSB, 1, SB), jnp.int32),
        pltpu.VMEM((E, DIM), jnp.float32),
        pltpu.VMEM((E, DIM), jnp.float32),
        pltpu.VMEM((CHUNK_ROWS, DIM), jnp.float32),
        pltpu.VMEM((1, DIM), jnp.float32),
        pltpu.SemaphoreType.DMA,
        pltpu.SemaphoreType.DMA,
    ],
)
def _sc_embed(idx_hbm, table_hbm, out_hbm,
              idxA, idxB, rowsA, rowsB, out_v, padrow, semA, semB):
    cid = lax.axis_index("c")
    sid = lax.axis_index("s")
    wid = cid * NS + sid

    iota = lax.iota(jnp.int32, 16)
    zero_i = jnp.zeros((16,), jnp.int32)
    one_i = jnp.full((16,), 1, jnp.int32)
    vocab_v = jnp.full((16,), VOCAB_N, jnp.int32)
    mclamp = [jnp.minimum(iota + mb, MVALS - 1) for mb in (0, 16, 32, 48)]
    mvalid = [(iota + mb) < MVALS for mb in (0, 16, 32, 48)]

    pltpu.sync_copy(table_hbm.at[pl.ds(VOCAB_N, 1)], padrow)
    pad0 = padrow[0, 0:16]
    pad1 = padrow[0, 16:32]
    padflag = jnp.minimum(
        plsc.all_reduce_population_count(jnp.maximum(pad0, pad1) > 0.0), one_i)

    def load_idx(ch, idx_v):
        sb_base = wid * (N_CHUNKS * NSB) + ch * NSB
        pltpu.sync_copy(idx_hbm.at[pl.ds(sb_base, NSB)], idx_v)

    def fire_gathers(idx_v, rows_v, sem):
        @pl.loop(0, NSB)
        def _fire(j):
            pltpu.async_copy(table_hbm.at[idx_v.at[j, 0]],
                             rows_v.at[pl.ds(j * SB, SB)], sem)

    def process(ch, idx_v, rows_v, sem):
        @pl.loop(0, NSB)
        def _drain(j):
            pltpu.make_async_copy(table_hbm.at[idx_v.at[j, 0]],
                                  rows_v.at[pl.ds(j * SB, SB)], sem).wait()

        @pl.loop(0, CHUNK_ROWS)
        def _row(r):
            r50 = jnp.full((16,), r * MVALS, jnp.int32)
            nval_vec = zero_i
            for grp in range(4):
                ridx = r50 + mclamp[grp]
                rhi = ridx >> 6
                rlo = ridx & 63
                idxs = plsc.load_gather(idx_v, [rhi, zero_i, rlo])
                valid = (idxs < vocab_v) & mvalid[grp]
                nval_vec = nval_vec + plsc.all_reduce_population_count(valid)
            e0 = r * MVALS
            s0 = rows_v[e0, 0:16]
            s1 = rows_v[e0, 16:32]
            any_vec = jnp.minimum(
                plsc.all_reduce_population_count(jnp.maximum(s0, s1) > 0.0),
                one_i)
            for m in range(1, MVALS):
                v0 = rows_v[e0 + m, 0:16]
                v1 = rows_v[e0 + m, 16:32]
                s0 = s0 + v0
                s1 = s1 + v1
                pc = plsc.all_reduce_population_count(
                    jnp.maximum(v0, v1) > 0.0)
                any_vec = any_vec + jnp.minimum(pc, one_i)
            npad_i = MVALS - nval_vec
            n_vec = any_vec - npad_i * padflag
            npad = npad_i.astype(jnp.float32)
            rec = 1.0 / jnp.maximum(n_vec.astype(jnp.float32), 1.0)
            out_v[r, 0:16] = (s0 - npad * pad0) * rec
            out_v[r, 16:32] = (s1 - npad * pad1) * rec

        rbase = wid * ROWS_PER_W + ch * CHUNK_ROWS
        pltpu.sync_copy(out_v, out_hbm.at[pl.ds(rbase, CHUNK_ROWS)])

    load_idx(0, idxA)
    fire_gathers(idxA, rowsA, semA)

    @pl.loop(0, N_CHUNKS // 2)
    def _pair(g):
        ch0 = 2 * g
        load_idx(ch0 + 1, idxB)
        fire_gathers(idxB, rowsB, semB)
        process(ch0, idxA, rowsA, semA)

        @pl.when(g < N_CHUNKS // 2 - 1)
        def _pf():
            load_idx(ch0 + 2, idxA)
            fire_gathers(idxA, rowsA, semA)

        process(ch0 + 1, idxB, rowsB, semB)


def kernel(indices, table):
    idx3 = indices.reshape(BATCH * MVALS // SB, 1, SB)
    out = _sc_embed(idx3, table)
    return out.reshape(BATCH, 1, DIM)


# speedup vs baseline: 1.7245x; 1.0002x over previous
import dataclasses
import functools

import jax
import jax.numpy as jnp
from jax import lax
from jax.experimental import pallas as pl
from jax.experimental.pallas import tpu as pltpu
from jax.experimental.pallas import tpu_sc as plsc

VOCAB_N = 1000000
DIM = 32
MVALS = 50
BATCH = 16384

NC = 2
NS = 16
NW = NC * NS
ROWS_PER_W = BATCH // NW
CHUNK_ROWS = 32
N_CHUNKS = ROWS_PER_W // CHUNK_ROWS
E = CHUNK_ROWS * MVALS
SB = 64
NSB = E // SB

_mesh = plsc.VectorSubcoreMesh(core_axis_name="c", subcore_axis_name="s")

_cp = pltpu.CompilerParams()
if "needs_layout_passes" in pltpu.CompilerParams.__dataclass_fields__:
    _cp = dataclasses.replace(_cp, needs_layout_passes=False)
if "use_tc_tiling_on_sc" in pltpu.CompilerParams.__dataclass_fields__:
    _cp = dataclasses.replace(_cp, use_tc_tiling_on_sc=False)


@functools.partial(
    pl.kernel,
    out_type=jax.ShapeDtypeStruct((BATCH, DIM), jnp.float32),
    mesh=_mesh,
    compiler_params=_cp,
    scratch_types=[
        pltpu.VMEM((NSB, 1, SB), jnp.int32),
        pltpu.VMEM((NSB, 1, SB), jnp.int32),
        pltpu.VMEM((E, DIM), jnp.float32),
        pltpu.VMEM((E, DIM), jnp.float32),
        pltpu.VMEM((CHUNK_ROWS, DIM), jnp.float32),
        pltpu.VMEM((CHUNK_ROWS, DIM), jnp.float32),
        pltpu.VMEM((1, DIM), jnp.float32),
        pltpu.SemaphoreType.DMA,
        pltpu.SemaphoreType.DMA,
        pltpu.SemaphoreType.DMA,
        pltpu.SemaphoreType.DMA,
    ],
)
def _sc_embed(idx_hbm, table_hbm, out_hbm,
              idxA, idxB, rowsA, rowsB, outA, outB, padrow,
              semA, semB, semOA, semOB):
    cid = lax.axis_index("c")
    sid = lax.axis_index("s")
    wid = cid * NS + sid

    iota = lax.iota(jnp.int32, 16)
    zero_i = jnp.zeros((16,), jnp.int32)
    one_i = jnp.full((16,), 1, jnp.int32)
    vocab_v = jnp.full((16,), VOCAB_N, jnp.int32)
    mclamp = [jnp.minimum(iota + mb, MVALS - 1) for mb in (0, 16, 32, 48)]
    mvalid = [(iota + mb) < MVALS for mb in (0, 16, 32, 48)]

    pltpu.sync_copy(table_hbm.at[pl.ds(VOCAB_N, 1)], padrow)
    pad0 = padrow[0, 0:16]
    pad1 = padrow[0, 16:32]
    padflag = jnp.minimum(
        plsc.all_reduce_population_count(jnp.maximum(pad0, pad1) > 0.0), one_i)

    def load_idx(ch, idx_v):
        sb_base = wid * (N_CHUNKS * NSB) + ch * NSB
        pltpu.sync_copy(idx_hbm.at[pl.ds(sb_base, NSB)], idx_v)

    def fire_gathers(idx_v, rows_v, sem):
        @pl.loop(0, NSB)
        def _fire(j):
            pltpu.async_copy(table_hbm.at[idx_v.at[j, 0]],
                             rows_v.at[pl.ds(j * SB, SB)], sem)

    def process(ch, idx_v, rows_v, sem, out_v, semO):
        @pl.when(ch >= 2)
        def _ow():
            pb = wid * ROWS_PER_W + (ch - 2) * CHUNK_ROWS
            pltpu.make_async_copy(out_v, out_hbm.at[pl.ds(pb, CHUNK_ROWS)],
                                  semO).wait()
        @pl.loop(0, NSB)
        def _drain(j):
            pltpu.make_async_copy(table_hbm.at[idx_v.at[j, 0]],
                                  rows_v.at[pl.ds(j * SB, SB)], sem).wait()

        @pl.loop(0, CHUNK_ROWS)
        def _row(r):
            r50 = jnp.full((16,), r * MVALS, jnp.int32)
            nval_vec = zero_i
            for grp in range(4):
                ridx = r50 + mclamp[grp]
                rhi = ridx >> 6
                rlo = ridx & 63
                idxs = plsc.load_gather(idx_v, [rhi, zero_i, rlo])
                valid = (idxs < vocab_v) & mvalid[grp]
                nval_vec = nval_vec + plsc.all_reduce_population_count(valid)
            e0 = r * MVALS
            s0 = rows_v[e0, 0:16]
            s1 = rows_v[e0, 16:32]
            any_vec = jnp.minimum(
                plsc.all_reduce_population_count(jnp.maximum(s0, s1) > 0.0),
                one_i)
            for m in range(1, MVALS):
                v0 = rows_v[e0 + m, 0:16]
                v1 = rows_v[e0 + m, 16:32]
                s0 = s0 + v0
                s1 = s1 + v1
                pc = plsc.all_reduce_population_count(
                    jnp.maximum(v0, v1) > 0.0)
                any_vec = any_vec + jnp.minimum(pc, one_i)
            npad_i = MVALS - nval_vec
            n_vec = any_vec - npad_i * padflag
            npad = npad_i.astype(jnp.float32)
            rec = 1.0 / jnp.maximum(n_vec.astype(jnp.float32), 1.0)
            out_v[r, 0:16] = (s0 - npad * pad0) * rec
            out_v[r, 16:32] = (s1 - npad * pad1) * rec

        rbase = wid * ROWS_PER_W + ch * CHUNK_ROWS
        pltpu.async_copy(out_v, out_hbm.at[pl.ds(rbase, CHUNK_ROWS)], semO)

    load_idx(0, idxA)
    fire_gathers(idxA, rowsA, semA)

    @pl.loop(0, N_CHUNKS // 2)
    def _pair(g):
        ch0 = 2 * g
        load_idx(ch0 + 1, idxB)
        fire_gathers(idxB, rowsB, semB)
        process(ch0, idxA, rowsA, semA, outA, semOA)

        @pl.when(g < N_CHUNKS // 2 - 1)
        def _pf():
            load_idx(ch0 + 2, idxA)
            fire_gathers(idxA, rowsA, semA)

        process(ch0 + 1, idxB, rowsB, semB, outB, semOB)

    pltpu.make_async_copy(
        outA, out_hbm.at[pl.ds(wid * ROWS_PER_W + (N_CHUNKS - 2) * CHUNK_ROWS,
                               CHUNK_ROWS)], semOA).wait()
    pltpu.make_async_copy(
        outB, out_hbm.at[pl.ds(wid * ROWS_PER_W + (N_CHUNKS - 1) * CHUNK_ROWS,
                               CHUNK_ROWS)], semOB).wait()


def kernel(indices, table):
    idx3 = indices.reshape(BATCH * MVALS // SB, 1, SB)
    out = _sc_embed(idx3, table)
    return out.reshape(BATCH, 1, DIM)


# speedup vs baseline: 1.7366x; 1.0070x over previous
import dataclasses
import functools

import jax
import jax.numpy as jnp
from jax import lax
from jax.experimental import pallas as pl
from jax.experimental.pallas import tpu as pltpu
from jax.experimental.pallas import tpu_sc as plsc

VOCAB_N = 1000000
DIM = 32
MVALS = 50
BATCH = 16384

NC = 2
NS = 16
NW = NC * NS
ROWS_PER_W = BATCH // NW
CHUNK_ROWS = 32
N_CHUNKS = ROWS_PER_W // CHUNK_ROWS
E = CHUNK_ROWS * MVALS
SB = 64
NSB = E // SB

_mesh = plsc.VectorSubcoreMesh(core_axis_name="c", subcore_axis_name="s")

_cp = pltpu.CompilerParams()
if "needs_layout_passes" in pltpu.CompilerParams.__dataclass_fields__:
    _cp = dataclasses.replace(_cp, needs_layout_passes=False)
if "use_tc_tiling_on_sc" in pltpu.CompilerParams.__dataclass_fields__:
    _cp = dataclasses.replace(_cp, use_tc_tiling_on_sc=False)


@functools.partial(
    pl.kernel,
    out_type=jax.ShapeDtypeStruct((BATCH, DIM), jnp.float32),
    mesh=_mesh,
    compiler_params=_cp,
    scratch_types=[
        pltpu.VMEM((E,), jnp.int32),
        pltpu.VMEM((E,), jnp.int32),
        pltpu.VMEM((E, DIM), jnp.float32),
        pltpu.VMEM((E, DIM), jnp.float32),
        pltpu.VMEM((CHUNK_ROWS, DIM), jnp.float32),
        pltpu.VMEM((CHUNK_ROWS, DIM), jnp.float32),
        pltpu.VMEM((1, DIM), jnp.float32),
        pltpu.SemaphoreType.DMA,
        pltpu.SemaphoreType.DMA,
        pltpu.SemaphoreType.DMA,
        pltpu.SemaphoreType.DMA,
    ],
)
def _sc_embed(idx_hbm, table_hbm, out_hbm,
              idxA, idxB, rowsA, rowsB, outA, outB, padrow,
              semA, semB, semOA, semOB):
    cid = lax.axis_index("c")
    sid = lax.axis_index("s")
    wid = cid * NS + sid

    iota = lax.iota(jnp.int32, 16)
    zero_i = jnp.zeros((16,), jnp.int32)
    one_i = jnp.full((16,), 1, jnp.int32)
    vocab_v = jnp.full((16,), VOCAB_N, jnp.int32)
    mclamp = [jnp.minimum(iota + mb, MVALS - 1) for mb in (0, 16, 32, 48)]
    mvalid = [(iota + mb) < MVALS for mb in (0, 16, 32, 48)]

    pltpu.sync_copy(table_hbm.at[pl.ds(VOCAB_N, 1)], padrow)
    pad0 = padrow[0, 0:16]
    pad1 = padrow[0, 16:32]
    padflag = jnp.minimum(
        plsc.all_reduce_population_count(jnp.maximum(pad0, pad1) > 0.0), one_i)

    def load_idx(ch, idx_v):
        base = wid * (N_CHUNKS * E) + ch * E
        pltpu.sync_copy(idx_hbm.at[pl.ds(base, E)], idx_v)

    def fire_gathers(idx_v, rows_v, sem):
        pltpu.async_copy(table_hbm.at[idx_v], rows_v, sem)

    def process(ch, idx_v, rows_v, sem, out_v, semO):
        @pl.when(ch >= 2)
        def _ow():
            pb = wid * ROWS_PER_W + (ch - 2) * CHUNK_ROWS
            pltpu.make_async_copy(out_v, out_hbm.at[pl.ds(pb, CHUNK_ROWS)],
                                  semO).wait()
        pltpu.make_async_copy(table_hbm.at[idx_v], rows_v, sem).wait()

        @pl.loop(0, CHUNK_ROWS)
        def _row(r):
            r50 = jnp.full((16,), r * MVALS, jnp.int32)
            nval_vec = zero_i
            for grp in range(4):
                ridx = r50 + mclamp[grp]
                idxs = plsc.load_gather(idx_v, [ridx])
                valid = (idxs < vocab_v) & mvalid[grp]
                nval_vec = nval_vec + plsc.all_reduce_population_count(valid)
            e0 = r * MVALS
            s0 = rows_v[e0, 0:16]
            s1 = rows_v[e0, 16:32]
            any_vec = jnp.minimum(
                plsc.all_reduce_population_count(jnp.maximum(s0, s1) > 0.0),
                one_i)
            for m in range(1, MVALS):
                v0 = rows_v[e0 + m, 0:16]
                v1 = rows_v[e0 + m, 16:32]
                s0 = s0 + v0
                s1 = s1 + v1
                pc = plsc.all_reduce_population_count(
                    jnp.maximum(v0, v1) > 0.0)
                any_vec = any_vec + jnp.minimum(pc, one_i)
            npad_i = MVALS - nval_vec
            n_vec = any_vec - npad_i * padflag
            npad = npad_i.astype(jnp.float32)
            rec = 1.0 / jnp.maximum(n_vec.astype(jnp.float32), 1.0)
            out_v[r, 0:16] = (s0 - npad * pad0) * rec
            out_v[r, 16:32] = (s1 - npad * pad1) * rec

        rbase = wid * ROWS_PER_W + ch * CHUNK_ROWS
        pltpu.async_copy(out_v, out_hbm.at[pl.ds(rbase, CHUNK_ROWS)], semO)

    load_idx(0, idxA)
    fire_gathers(idxA, rowsA, semA)

    @pl.loop(0, N_CHUNKS // 2)
    def _pair(g):
        ch0 = 2 * g
        load_idx(ch0 + 1, idxB)
        fire_gathers(idxB, rowsB, semB)
        process(ch0, idxA, rowsA, semA, outA, semOA)

        @pl.when(g < N_CHUNKS // 2 - 1)
        def _pf():
            load_idx(ch0 + 2, idxA)
            fire_gathers(idxA, rowsA, semA)

        process(ch0 + 1, idxB, rowsB, semB, outB, semOB)

    pltpu.make_async_copy(
        outA, out_hbm.at[pl.ds(wid * ROWS_PER_W + (N_CHUNKS - 2) * CHUNK_ROWS,
                               CHUNK_ROWS)], semOA).wait()
    pltpu.make_async_copy(
        outB, out_hbm.at[pl.ds(wid * ROWS_PER_W + (N_CHUNKS - 1) * CHUNK_ROWS,
                               CHUNK_ROWS)], semOB).wait()


def kernel(indices, table):
    idx_flat = indices.reshape(BATCH * MVALS)
    out = _sc_embed(idx_flat, table)
    return out.reshape(BATCH, 1, DIM)
